# baseline (speedup 1.0000x reference)
import functools

import jax
import jax.numpy as jnp
from jax import lax
from jax.experimental import pallas as pl
from jax.experimental.pallas import tpu as pltpu
from jax.experimental.pallas import tpu_sc as plsc

N = 10000
E = 320000
D = 128

NC = 2
NS = 16
NW = NC * NS
EPW = E // NW
K = 50
NCHUNK = EPW // K
G = 10
NG = NCHUNK // G
KP = 64
R = 5
NPAD = 10240
SHARE = NPAD // NS

_sc_mesh = plsc.VectorSubcoreMesh(core_axis_name="c", subcore_axis_name="s")

_dnums = lax.GatherDimensionNumbers(offset_dims=(),
                                    collapsed_slice_dims=(0,),
                                    start_index_map=(0,))


@functools.partial(
    pl.kernel,
    out_type=jax.ShapeDtypeStruct((NC, NPAD, D), jnp.float32),
    mesh=_sc_mesh,
    scratch_types=[
        pltpu.VMEM((2, G, 2, K), jnp.int32),
        pltpu.VMEM((2, G, 1, KP), jnp.float32),
        pltpu.VMEM((R, K, D), jnp.float32),
        pltpu.VMEM_SHARED((NPAD, D), jnp.float32),
        pltpu.SemaphoreType.DMA((2,)),
        pltpu.SemaphoreType.DMA((R,)),
        pltpu.SemaphoreType.DMA((R,)),
    ],
)
def _edge_agg(m_hbm, pk_hbm, w_hbm, z_hbm, out_hbm,
              idxr, wr, rows_v, acc_sh, isem, gsem, ssem):
    c = lax.axis_index("c")
    s = lax.axis_index("s")
    wid = c * NS + s

    def idx_start(gg, sl):
        pltpu.async_copy(pk_hbm.at[wid, gg], idxr.at[sl], isem.at[sl])
        pltpu.async_copy(w_hbm.at[wid, gg], wr.at[sl], isem.at[sl])

    def idx_wait(sl):
        pltpu.make_async_copy(pk_hbm.at[wid, 0], idxr.at[sl],
                              isem.at[sl]).wait()
        pltpu.make_async_copy(w_hbm.at[wid, 0], wr.at[sl],
                              isem.at[sl]).wait()

    def gather_start(sl, k, rb):
        pltpu.async_copy(m_hbm.at[idxr.at[sl, k, 0]], rows_v.at[rb],
                         gsem.at[rb])

    def gather_wait(sl, k, rb):
        pltpu.make_async_copy(m_hbm.at[idxr.at[sl, k, 0]], rows_v.at[rb],
                              gsem.at[rb]).wait()

    def scatter_start(sl, k, rb):
        pltpu.async_copy(rows_v.at[rb], acc_sh.at[idxr.at[sl, k, 1]],
                         ssem.at[rb], add=True)

    def scatter_wait(sl, k, rb):
        pltpu.make_async_copy(rows_v.at[rb], acc_sh.at[idxr.at[sl, k, 1]],
                              ssem.at[rb]).wait()

    def do_chunk(cc, k, sl, rb, head, tail, prefetch_gg):
        gather_wait(sl, k, rb)
        if not head:
            if k >= 3:
                scatter_wait(sl, k - 3, (rb - 3) % R)
            else:
                scatter_wait(sl ^ 1, k - 3 + G, (rb - 3) % R)
        if not tail and k == G - 2:
            idx_wait(sl ^ 1)
        if not tail or k < G - 2:
            if k + 2 <= G - 1:
                gather_start(sl, k + 2, (rb + 2) % R)
            else:
                gather_start(sl ^ 1, k + 2 - G, (rb + 2) % R)
        if prefetch_gg is not None and k == 2:
            idx_start(prefetch_gg, sl ^ 1)

        @plsc.parallel_loop(0, K, step=1)
        def _edge(e):
            off = (e // 16) * 16
            w16 = wr[sl, k, 0, pl.ds(off, 16)]
            wspl = lax.gather(
                w16, jnp.full((16, 1), e - off, jnp.int32), _dnums, (1,),
                mode=lax.GatherScatterMode.PROMISE_IN_BOUNDS)
            for d in range(D // 16):
                sl_ = pl.ds(d * 16, 16)
                rows_v[rb, e, sl_] = rows_v[rb, e, sl_] * wspl

        scatter_start(sl, k, rb)

    idx_start(0, 0)
    pltpu.sync_copy(z_hbm, acc_sh.at[pl.ds(s * SHARE, SHARE)])
    idx_wait(0)
    gather_start(0, 0, 0)
    gather_start(0, 1, 1)
    plsc.subcore_barrier()

    for k in range(G):
        do_chunk(k, k, 0, k % R, k < 3, False, 1 if k == 2 else None)
    for k in range(G):
        do_chunk(G + k, k, 1, (G + k) % R, False, False,
                 2 if k == 2 else None)

    def pair(gp, carry):
        cc0 = gp * 2 * G
        for k in range(G):
            do_chunk(cc0 + k, k, 0, k % R, False, False,
                     2 * gp + 1 if k == 2 else None)
        for k in range(G):
            do_chunk(cc0 + G + k, k, 1, (G + k) % R, False, False,
                     2 * gp + 2 if k == 2 else None)
        return carry

    lax.fori_loop(1, (NG - 2) // 2, pair, 0)

    cc0 = (NG - 2) * G
    for k in range(G):
        do_chunk(cc0 + k, k, 0, k % R, False, False,
                 NG - 1 if k == 2 else None)
    for k in range(G):
        do_chunk(cc0 + G + k, k, 1, (G + k) % R, False, True, None)
    for k in range(G - 3, G):
        scatter_wait(1, k, (cc0 + G + k) % R)

    plsc.subcore_barrier()
    pltpu.sync_copy(acc_sh.at[pl.ds(s * SHARE, SHARE)],
                    out_hbm.at[c, pl.ds(s * SHARE, SHARE)])


_R = 1000


def _dot(a, b, dims):
    return lax.dot_general(a, b, (dims, ((), ())),
                           preferred_element_type=jnp.float32)


def _mm_body(x_ref, w_ref, o_ref):
    o_ref[...] = _dot(x_ref[...], w_ref[...], ((1,), (0,)))


def _matmul(x, w):
    return pl.pallas_call(
        _mm_body,
        grid=(N // _R,),
        in_specs=[pl.BlockSpec((_R, D), lambda i: (i, 0)),
                  pl.BlockSpec((D, D), lambda i: (0, 0))],
        out_specs=pl.BlockSpec((_R, D), lambda i: (i, 0)),
        out_shape=jax.ShapeDtypeStruct((N, D), jnp.float32),
    )(x, w)


def _gru_compute(aggA, aggB, h, wih, whh, bih, bhh):
    agg = aggA + aggB
    gi = _dot(agg, wih, ((1,), (1,))) + bih
    gh = _dot(h, whh, ((1,), (1,))) + bhh
    r = jax.nn.sigmoid(gi[:, :D] + gh[:, :D])
    z = jax.nn.sigmoid(gi[:, D:2 * D] + gh[:, D:2 * D])
    n = jnp.tanh(gi[:, 2 * D:] + r * gh[:, 2 * D:])
    return (1.0 - z) * n + z * h


def _gru_mm_body(aggA_ref, aggB_ref, h_ref, wih_ref, whh_ref, bih_ref,
                 bhh_ref, wg_ref, hout_ref, mout_ref):
    hn = _gru_compute(aggA_ref[...], aggB_ref[...], h_ref[...],
                      wih_ref[...], whh_ref[...], bih_ref[...], bhh_ref[...])
    hout_ref[...] = hn
    mout_ref[...] = _dot(hn, wg_ref[...], ((1,), (0,)))


def _gru_mm(aggA, aggB, h, wih, whh, bih, bhh, wg):
    row = pl.BlockSpec((_R, D), lambda i: (i, 0))
    return pl.pallas_call(
        _gru_mm_body,
        grid=(N // _R,),
        in_specs=[row, row, row,
                  pl.BlockSpec((3 * D, D), lambda i: (0, 0)),
                  pl.BlockSpec((3 * D, D), lambda i: (0, 0)),
                  pl.BlockSpec((1, 3 * D), lambda i: (0, 0)),
                  pl.BlockSpec((1, 3 * D), lambda i: (0, 0)),
                  pl.BlockSpec((D, D), lambda i: (0, 0))],
        out_specs=[row, row],
        out_shape=[jax.ShapeDtypeStruct((N, D), jnp.float32),
                   jax.ShapeDtypeStruct((N, D), jnp.float32)],
    )(aggA, aggB, h, wih, whh, bih, bhh, wg)


def _gru_lstm_body(aggA_ref, aggB_ref, h_ref, wih_ref, whh_ref, bih_ref,
                   bhh_ref, lwih_ref, lbih_ref, lbhh_ref, hout_ref, cout_ref):
    hn = _gru_compute(aggA_ref[...], aggB_ref[...], h_ref[...],
                      wih_ref[...], whh_ref[...], bih_ref[...], bhh_ref[...])
    gates = _dot(hn, lwih_ref[...], ((1,), (1,))) + lbih_ref[...] + lbhh_ref[...]
    i_t = jax.nn.sigmoid(gates[:, :D])
    g_t = jnp.tanh(gates[:, 2 * D:3 * D])
    o_t = jax.nn.sigmoid(gates[:, 3 * D:])
    c_t = i_t * g_t
    hout_ref[...] = o_t * jnp.tanh(c_t)
    cout_ref[...] = c_t


def _gru_lstm(aggA, aggB, h, wih, whh, bih, bhh, lwih, lbih, lbhh):
    row = pl.BlockSpec((_R, D), lambda i: (i, 0))
    return pl.pallas_call(
        _gru_lstm_body,
        grid=(N // _R,),
        in_specs=[row, row, row,
                  pl.BlockSpec((3 * D, D), lambda i: (0, 0)),
                  pl.BlockSpec((3 * D, D), lambda i: (0, 0)),
                  pl.BlockSpec((1, 3 * D), lambda i: (0, 0)),
                  pl.BlockSpec((1, 3 * D), lambda i: (0, 0)),
                  pl.BlockSpec((4 * D, D), lambda i: (0, 0)),
                  pl.BlockSpec((1, 4 * D), lambda i: (0, 0)),
                  pl.BlockSpec((1, 4 * D), lambda i: (0, 0))],
        out_specs=[row, row],
        out_shape=[jax.ShapeDtypeStruct((N, D), jnp.float32),
                   jax.ShapeDtypeStruct((N, D), jnp.float32)],
    )(aggA, aggB, h, wih, whh, bih, bhh, lwih, lbih, lbhh)


def kernel(X, edge_index, edge_weight, ggc_weight,
           gru_w_ih, gru_w_hh, gru_b_ih, gru_b_hh,
           lstm_w_ih, lstm_w_hh, lstm_b_ih, lstm_b_hh):
    src5 = edge_index[0].reshape(NW, NG, G, 1, K)
    dst5 = edge_index[1].reshape(NW, NG, G, 1, K)
    pk = jnp.concatenate([src5, dst5], axis=3)
    w5 = jnp.pad(edge_weight.reshape(NW, NG, G, 1, K),
                 ((0, 0), (0, 0), (0, 0), (0, 0), (0, KP - K)))
    z = jnp.zeros((SHARE, D), jnp.float32)
    bih = gru_b_ih.reshape(1, 3 * D)
    bhh = gru_b_hh.reshape(1, 3 * D)
    lbih = lstm_b_ih.reshape(1, 4 * D)
    lbhh = lstm_b_hh.reshape(1, 4 * D)

    m = _matmul(X, ggc_weight[0])
    parts = _edge_agg(m, pk, w5, z)
    h1, m1 = _gru_mm(parts[0, :N], parts[1, :N], X,
                     gru_w_ih, gru_w_hh, bih, bhh, ggc_weight[1])
    parts = _edge_agg(m1, pk, w5, z)
    H, C = _gru_lstm(parts[0, :N], parts[1, :N], h1,
                     gru_w_ih, gru_w_hh, bih, bhh, lstm_w_ih, lbih, lbhh)
    return (H, H, C)


# speedup vs baseline: 1.0243x; 1.0243x over previous
import functools

import jax
import jax.numpy as jnp
from jax import lax
from jax.experimental import pallas as pl
from jax.experimental.pallas import tpu as pltpu
from jax.experimental.pallas import tpu_sc as plsc

N = 10000
E = 320000
D = 128

NC = 2
NS = 16
NW = NC * NS
EPW = E // NW
K = 100
NCHUNK = EPW // K
G = 10
NG = NCHUNK // G
KP = 112
NPAD = 10240
SHARE = NPAD // NS

_sc_mesh = plsc.VectorSubcoreMesh(core_axis_name="c", subcore_axis_name="s")

_dnums = lax.GatherDimensionNumbers(offset_dims=(),
                                    collapsed_slice_dims=(0,),
                                    start_index_map=(0,))


@functools.partial(
    pl.kernel,
    out_type=jax.ShapeDtypeStruct((NC, NPAD, D), jnp.float32),
    mesh=_sc_mesh,
    scratch_types=[
        pltpu.VMEM((2, G, 2, K), jnp.int32),
        pltpu.VMEM((2, G, 1, KP), jnp.float32),
        pltpu.VMEM((2, K, D), jnp.float32),
        pltpu.VMEM_SHARED((NPAD, D), jnp.float32),
        pltpu.SemaphoreType.DMA((2,)),
        pltpu.SemaphoreType.DMA((2,)),
        pltpu.SemaphoreType.DMA((2,)),
    ],
)
def _edge_agg(m_hbm, pk_hbm, w_hbm, z_hbm, out_hbm,
              idxr, wr, rows_v, acc_sh, isem, gsem, ssem):
    c = lax.axis_index("c")
    s = lax.axis_index("s")
    wid = c * NS + s

    def idx_start(gg, sl):
        pltpu.async_copy(pk_hbm.at[wid, gg], idxr.at[sl], isem.at[sl])
        pltpu.async_copy(w_hbm.at[wid, gg], wr.at[sl], isem.at[sl])

    def idx_wait(sl):
        pltpu.make_async_copy(pk_hbm.at[wid, 0], idxr.at[sl],
                              isem.at[sl]).wait()
        pltpu.make_async_copy(w_hbm.at[wid, 0], wr.at[sl],
                              isem.at[sl]).wait()

    def gather_start(sl, k, rb):
        pltpu.async_copy(m_hbm.at[idxr.at[sl, k, 0]], rows_v.at[rb],
                         gsem.at[rb])

    def gather_wait(sl, k, rb):
        pltpu.make_async_copy(m_hbm.at[idxr.at[sl, k, 0]], rows_v.at[rb],
                              gsem.at[rb]).wait()

    def scatter_start(sl, k, rb):
        pltpu.async_copy(rows_v.at[rb], acc_sh.at[idxr.at[sl, k, 1]],
                         ssem.at[rb], add=True)

    def scatter_wait(sl, k, rb):
        pltpu.make_async_copy(rows_v.at[rb], acc_sh.at[idxr.at[sl, k, 1]],
                              ssem.at[rb]).wait()

    def do_chunk(gg, k, sl, first, last, prefetch):
        rb = k % 2
        gather_wait(sl, k, rb)
        if not last:
            if not first:
                pk_, ps = (k - 1, sl) if k > 0 else (G - 1, sl ^ 1)
                scatter_wait(ps, pk_, rb ^ 1)
            if k == G - 1:
                idx_wait(sl ^ 1)
                gather_start(sl ^ 1, 0, rb ^ 1)
            else:
                gather_start(sl, k + 1, rb ^ 1)
        if prefetch is not None and k == 0:
            idx_start(prefetch, sl ^ 1)

        @plsc.parallel_loop(0, K, step=1, unroll=2)
        def _edge(e):
            off = (e // 16) * 16
            w16 = wr[sl, k, 0, pl.ds(off, 16)]
            wspl = lax.gather(
                w16, jnp.full((16, 1), e - off, jnp.int32), _dnums, (1,),
                mode=lax.GatherScatterMode.PROMISE_IN_BOUNDS)
            for d in range(D // 16):
                sl_ = pl.ds(d * 16, 16)
                rows_v[rb, e, sl_] = rows_v[rb, e, sl_] * wspl

        scatter_start(sl, k, rb)

    idx_start(0, 0)
    pltpu.sync_copy(z_hbm, acc_sh.at[pl.ds(s * SHARE, SHARE)])
    idx_wait(0)
    gather_start(0, 0, 0)
    plsc.subcore_barrier()

    for k in range(G):
        do_chunk(0, k, 0, k == 0, False, 1 if k == 0 else None)

    def pair(gp, carry):
        g1 = 1 + 2 * gp
        for k in range(G):
            do_chunk(g1, k, 1, False, False, g1 + 1 if k == 0 else None)
        for k in range(G):
            do_chunk(g1 + 1, k, 0, False, False, g1 + 2 if k == 0 else None)
        return carry

    lax.fori_loop(0, (NG - 2) // 2, pair, 0)

    for k in range(G):
        do_chunk(NG - 1, k, (NG - 1) % 2, False, k == G - 1, None)
    scatter_wait((NG - 1) % 2, G - 2, (G - 2) % 2)
    scatter_wait((NG - 1) % 2, G - 1, (G - 1) % 2)

    plsc.subcore_barrier()
    pltpu.sync_copy(acc_sh.at[pl.ds(s * SHARE, SHARE)],
                    out_hbm.at[c, pl.ds(s * SHARE, SHARE)])


_R = 2000


def _dot(a, b, dims):
    return lax.dot_general(a, b, (dims, ((), ())),
                           preferred_element_type=jnp.float32)


def _mm_body(x_ref, w_ref, o_ref):
    o_ref[...] = _dot(x_ref[...], w_ref[...], ((1,), (0,)))


def _matmul(x, w):
    return pl.pallas_call(
        _mm_body,
        grid=(N // _R,),
        in_specs=[pl.BlockSpec((_R, D), lambda i: (i, 0)),
                  pl.BlockSpec((D, D), lambda i: (0, 0))],
        out_specs=pl.BlockSpec((_R, D), lambda i: (i, 0)),
        out_shape=jax.ShapeDtypeStruct((N, D), jnp.float32),
    )(x, w)


def _gru_compute(aggA, aggB, h, wih, whh, bih, bhh):
    agg = aggA + aggB
    gi = _dot(agg, wih, ((1,), (1,))) + bih
    gh = _dot(h, whh, ((1,), (1,))) + bhh
    r = jax.nn.sigmoid(gi[:, :D] + gh[:, :D])
    z = jax.nn.sigmoid(gi[:, D:2 * D] + gh[:, D:2 * D])
    n = jnp.tanh(gi[:, 2 * D:] + r * gh[:, 2 * D:])
    return (1.0 - z) * n + z * h


def _gru_mm_body(aggA_ref, aggB_ref, h_ref, wih_ref, whh_ref, bih_ref,
                 bhh_ref, wg_ref, hout_ref, mout_ref):
    hn = _gru_compute(aggA_ref[...], aggB_ref[...], h_ref[...],
                      wih_ref[...], whh_ref[...], bih_ref[...], bhh_ref[...])
    hout_ref[...] = hn
    mout_ref[...] = _dot(hn, wg_ref[...], ((1,), (0,)))


def _gru_mm(aggA, aggB, h, wih, whh, bih, bhh, wg):
    row = pl.BlockSpec((_R, D), lambda i: (i, 0))
    return pl.pallas_call(
        _gru_mm_body,
        grid=(N // _R,),
        in_specs=[row, row, row,
                  pl.BlockSpec((3 * D, D), lambda i: (0, 0)),
                  pl.BlockSpec((3 * D, D), lambda i: (0, 0)),
                  pl.BlockSpec((1, 3 * D), lambda i: (0, 0)),
                  pl.BlockSpec((1, 3 * D), lambda i: (0, 0)),
                  pl.BlockSpec((D, D), lambda i: (0, 0))],
        out_specs=[row, row],
        out_shape=[jax.ShapeDtypeStruct((N, D), jnp.float32),
                   jax.ShapeDtypeStruct((N, D), jnp.float32)],
    )(aggA, aggB, h, wih, whh, bih, bhh, wg)


def _gru_lstm_body(aggA_ref, aggB_ref, h_ref, wih_ref, whh_ref, bih_ref,
                   bhh_ref, lwih_ref, lbih_ref, lbhh_ref, hout_ref, cout_ref):
    hn = _gru_compute(aggA_ref[...], aggB_ref[...], h_ref[...],
                      wih_ref[...], whh_ref[...], bih_ref[...], bhh_ref[...])
    gates = _dot(hn, lwih_ref[...], ((1,), (1,))) + lbih_ref[...] + lbhh_ref[...]
    i_t = jax.nn.sigmoid(gates[:, :D])
    g_t = jnp.tanh(gates[:, 2 * D:3 * D])
    o_t = jax.nn.sigmoid(gates[:, 3 * D:])
    c_t = i_t * g_t
    hout_ref[...] = o_t * jnp.tanh(c_t)
    cout_ref[...] = c_t


def _gru_lstm(aggA, aggB, h, wih, whh, bih, bhh, lwih, lbih, lbhh):
    row = pl.BlockSpec((_R, D), lambda i: (i, 0))
    return pl.pallas_call(
        _gru_lstm_body,
        grid=(N // _R,),
        in_specs=[row, row, row,
                  pl.BlockSpec((3 * D, D), lambda i: (0, 0)),
                  pl.BlockSpec((3 * D, D), lambda i: (0, 0)),
                  pl.BlockSpec((1, 3 * D), lambda i: (0, 0)),
                  pl.BlockSpec((1, 3 * D), lambda i: (0, 0)),
                  pl.BlockSpec((4 * D, D), lambda i: (0, 0)),
                  pl.BlockSpec((1, 4 * D), lambda i: (0, 0)),
                  pl.BlockSpec((1, 4 * D), lambda i: (0, 0))],
        out_specs=[row, row],
        out_shape=[jax.ShapeDtypeStruct((N, D), jnp.float32),
                   jax.ShapeDtypeStruct((N, D), jnp.float32)],
    )(aggA, aggB, h, wih, whh, bih, bhh, lwih, lbih, lbhh)


def kernel(X, edge_index, edge_weight, ggc_weight,
           gru_w_ih, gru_w_hh, gru_b_ih, gru_b_hh,
           lstm_w_ih, lstm_w_hh, lstm_b_ih, lstm_b_hh):
    src5 = edge_index[0].reshape(NW, NG, G, 1, K)
    dst5 = edge_index[1].reshape(NW, NG, G, 1, K)
    pk = jnp.concatenate([src5, dst5], axis=3)
    w5 = jnp.pad(edge_weight.reshape(NW, NG, G, 1, K),
                 ((0, 0), (0, 0), (0, 0), (0, 0), (0, KP - K)))
    z = jnp.zeros((SHARE, D), jnp.float32)
    bih = gru_b_ih.reshape(1, 3 * D)
    bhh = gru_b_hh.reshape(1, 3 * D)
    lbih = lstm_b_ih.reshape(1, 4 * D)
    lbhh = lstm_b_hh.reshape(1, 4 * D)

    m = _matmul(X, ggc_weight[0])
    parts = _edge_agg(m, pk, w5, z)
    h1, m1 = _gru_mm(parts[0, :N], parts[1, :N], X,
                     gru_w_ih, gru_w_hh, bih, bhh, ggc_weight[1])
    parts = _edge_agg(m1, pk, w5, z)
    H, C = _gru_lstm(parts[0, :N], parts[1, :N], h1,
                     gru_w_ih, gru_w_hh, bih, bhh, lstm_w_ih, lbih, lbhh)
    return (H, H, C)
